# baseline (speedup 1.0000x reference)
import functools

import jax
import jax.numpy as jnp
from jax import lax
from jax.experimental import pallas as pl
from jax.experimental.pallas import tpu as pltpu
from jax.experimental.pallas import tpu_sc as plsc

ALPHA = 0.2
NT = 10240
R1 = 264
CH = 2048
DSUB = 4
BN = 2048
D = 128


@functools.lru_cache(maxsize=None)
def _build_sc_edge_kernel(nchn: int, nchh: int):
  mesh = plsc.VectorSubcoreMesh(core_axis_name="c", subcore_axis_name="s",
                                num_cores=2, num_subcores=16)

  @functools.partial(
      pl.kernel,
      out_type=(jax.ShapeDtypeStruct((2, 16, NT * DSUB), jnp.float32),
                jax.ShapeDtypeStruct((2, NT), jnp.float32)),
      mesh=mesh,
      scratch_types=[
          pltpu.VMEM((NT,), jnp.float32),
          pltpu.VMEM((NT,), jnp.float32),
          pltpu.VMEM((R1,), jnp.float32),
          pltpu.VMEM((NT * 2,), jnp.int32),
          pltpu.VMEM((R1 * 2,), jnp.int32),
          pltpu.VMEM((NT * DSUB,), jnp.float32),
          pltpu.VMEM((NT,), jnp.float32),
          pltpu.VMEM((4 * CH,), jnp.int32),
          pltpu.VMEM((4 * CH,), jnp.int32),
          pltpu.VMEM((4 * CH,), jnp.int32),
          pltpu.SemaphoreType.DMA,
          pltpu.SemaphoreType.DMA,
          pltpu.SemaphoreType.DMA,
      ],
      compiler_params=pltpu.CompilerParams(needs_layout_passes=False))
  def sc_edge(packn_h, packh_h, as_h, ad_h, ar_h, xd_h, re_h,
              acc_o, rs_o,
              as_v, ad_v, ar_v, xd_v, re_v, acc_v, rs_v,
              bufA, bufB, bufC, semA, semB, semC):
    c = lax.axis_index("c")
    s = lax.axis_index("s")

    pltpu.sync_copy(as_h.at[c], as_v)
    pltpu.sync_copy(ad_h.at[c], ad_v)
    pltpu.sync_copy(ar_h.at[c], ar_v)
    pltpu.sync_copy(xd_h.at[c, s], xd_v)
    pltpu.sync_copy(re_h.at[c, s], re_v)

    zero16 = jnp.zeros((16,), jnp.float32)

    def zacc(i, carry):
      for k in range(8):
        acc_v[pl.ds(i * 128 + k * 16, 16)] = zero16
      return carry

    lax.fori_loop(0, NT * DSUB // 128, zacc, 0)

    def zrs(i, carry):
      for k in range(8):
        rs_v[pl.ds(i * 128 + k * 16, 16)] = zero16
      return carry

    lax.fori_loop(0, NT // 128, zrs, 0)

    rs_mask = jnp.broadcast_to(s == 0, (16,))

    def do_group(buf, gb, with_tb):
      e0 = buf[pl.ds(gb, 16)]
      e1 = buf[pl.ds(CH + gb, 16)]
      tA = buf[pl.ds(2 * CH + gb, 16)]
      a = (plsc.load_gather(as_v, [e0]) + plsc.load_gather(ad_v, [e1])
           + plsc.load_gather(ar_v, [tA]))
      if with_tb:
        tB = buf[pl.ds(3 * CH + gb, 16)]
        a = a + plsc.load_gather(ar_v, [tB])
      ee = jnp.exp(jnp.where(a > 0, -a, (-ALPHA) * a))
      plsc.addupdate_scatter(rs_v, [e0], ee, mask=rs_mask)
      himask = jnp.full((16,), -65536, jnp.int32)
      for p in range(2):
        gx = plsc.load_gather(xd_v, [e1 + p * NT])
        gr = plsc.load_gather(re_v, [tA + p * R1])
        v0 = (plsc.bitcast(gx << 16, jnp.float32)
              + plsc.bitcast(gr << 16, jnp.float32))
        v1 = (plsc.bitcast(gx & himask, jnp.float32)
              + plsc.bitcast(gr & himask, jnp.float32))
        if with_tb:
          gb = plsc.load_gather(re_v, [tB + p * R1])
          v0 = v0 + plsc.bitcast(gb << 16, jnp.float32)
          v1 = v1 + plsc.bitcast(gb & himask, jnp.float32)
        plsc.addupdate_scatter(acc_v, [e0 + (2 * p) * NT], ee * v0)
        plsc.addupdate_scatter(acc_v, [e0 + (2 * p + 1) * NT], ee * v1)

    def process_chunk(buf, with_tb):
      @plsc.parallel_loop(0, CH // 16, 1, unroll=4)
      def grp(g):
        do_group(buf, g * 16, with_tb)

    def run_segment(pack_h, nch, wpc, with_tb):
      def issue(buf, sem, ci):
        pltpu.async_copy(pack_h.at[pl.ds(ci * wpc, wpc)],
                         buf.at[pl.ds(0, wpc)], sem)

      def drain(buf, sem):
        pltpu.make_async_copy(pack_h.at[pl.ds(0, wpc)],
                              buf.at[pl.ds(0, wpc)], sem).wait()

      issue(bufA, semA, 0)
      issue(bufB, semB, 1)

      def body(i, carry):
        c = 3 * i
        drain(bufA, semA)
        process_chunk(bufA, with_tb)
        issue(bufC, semC, c + 2)
        drain(bufB, semB)
        process_chunk(bufB, with_tb)
        issue(bufA, semA, c + 3)
        drain(bufC, semC)
        process_chunk(bufC, with_tb)
        issue(bufB, semB, c + 4)
        return carry

      lax.fori_loop(0, nch // 3, body, 0)
      drain(bufA, semA)
      drain(bufB, semB)

    run_segment(packn_h, nchn, 3 * CH, False)
    run_segment(packh_h, nchh, 4 * CH, True)

    pltpu.sync_copy(acc_v, acc_o.at[c, s])

    @pl.when(s == 0)
    def _():
      pltpu.sync_copy(rs_v, rs_o.at[c])

  return sc_edge


def _full_spec(shape):
  return pl.BlockSpec(shape, lambda i: tuple(0 for _ in shape))


def _k1_body(x_ref, wst_ref, wdt_ref, w2_ref, xs_ref, xd_ref, as_ref, ad_ref):
  x = x_ref[...]
  xs = jnp.dot(x, wst_ref[...], preferred_element_type=jnp.float32)
  xd = jnp.dot(x, wdt_ref[...], preferred_element_type=jnp.float32)
  xs_ref[...] = xs
  xd_ref[...] = xd
  w2 = w2_ref[...]
  as_ref[...] = jnp.dot(xs, w2, preferred_element_type=jnp.float32)
  ad_ref[...] = jnp.dot(xd, w2, preferred_element_type=jnp.float32)


_k1 = pl.pallas_call(
    _k1_body,
    grid=(NT // BN,),
    in_specs=[
        pl.BlockSpec((BN, D), lambda i: (i, 0)),
        _full_spec((D, D)),
        _full_spec((D, D)),
        _full_spec((D, 8)),
    ],
    out_specs=[
        pl.BlockSpec((BN, D), lambda i: (i, 0)),
        pl.BlockSpec((BN, D), lambda i: (i, 0)),
        pl.BlockSpec((BN, 8), lambda i: (i, 0)),
        pl.BlockSpec((BN, 8), lambda i: (i, 0)),
    ],
    out_shape=[
        jax.ShapeDtypeStruct((NT, D), jnp.float32),
        jax.ShapeDtypeStruct((NT, D), jnp.float32),
        jax.ShapeDtypeStruct((NT, 8), jnp.float32),
        jax.ShapeDtypeStruct((NT, 8), jnp.float32),
    ],
)


def _rel_body(rel_ref, wrt1_ref, w2blk_ref, wr_ref, wrot_ref, w2o_ref,
              re1_ref, ar1_ref, outrel_ref, re2_ref, ar2_ref):
  rel = rel_ref[...]
  re1 = jnp.dot(rel, wrt1_ref[...], preferred_element_type=jnp.float32)
  re1_ref[...] = re1
  ar1_ref[...] = jnp.dot(re1, w2blk_ref[...], preferred_element_type=jnp.float32)
  outrel = jnp.dot(rel, wr_ref[...], preferred_element_type=jnp.float32)
  outrel_ref[...] = outrel
  re2 = jnp.dot(outrel, wrot_ref[...], preferred_element_type=jnp.float32)
  re2_ref[...] = re2
  ar2_ref[...] = jnp.dot(re2, w2o_ref[...], preferred_element_type=jnp.float32)


_krel = pl.pallas_call(
    _rel_body,
    grid=(1,),
    in_specs=[
        _full_spec((R1, D)),
        _full_spec((D, D)),
        _full_spec((D, 8)),
        _full_spec((D, D)),
        _full_spec((D, D)),
        _full_spec((D, 8)),
    ],
    out_specs=[
        _full_spec((R1, D)),
        _full_spec((R1, 8)),
        _full_spec((R1, D)),
        _full_spec((R1, D)),
        _full_spec((R1, 8)),
    ],
    out_shape=[
        jax.ShapeDtypeStruct((R1, D), jnp.float32),
        jax.ShapeDtypeStruct((R1, 8), jnp.float32),
        jax.ShapeDtypeStruct((R1, D), jnp.float32),
        jax.ShapeDtypeStruct((R1, D), jnp.float32),
        jax.ShapeDtypeStruct((R1, 8), jnp.float32),
    ],
)


def _k2_body(acc_ref, rs_ref, xs_ref, sel_ref, wsot_ref, wdot_ref, w2o_ref,
             xs2_ref, xd2_ref, as2_ref, ad2_ref):
  rs128 = jnp.dot(rs_ref[...], sel_ref[...], preferred_element_type=jnp.float32)
  lx = (acc_ref[...] + rs128 * xs_ref[...]) / jnp.maximum(rs128, 1e-12)
  xs2 = jnp.dot(lx, wsot_ref[...], preferred_element_type=jnp.float32)
  xd2 = jnp.dot(lx, wdot_ref[...], preferred_element_type=jnp.float32)
  xs2_ref[...] = xs2
  xd2_ref[...] = xd2
  w2o = w2o_ref[...]
  as2_ref[...] = jnp.dot(xs2, w2o, preferred_element_type=jnp.float32)
  ad2_ref[...] = jnp.dot(xd2, w2o, preferred_element_type=jnp.float32)


_k2 = pl.pallas_call(
    _k2_body,
    grid=(NT // BN,),
    in_specs=[
        pl.BlockSpec((BN, D), lambda i: (i, 0)),
        pl.BlockSpec((BN, 8), lambda i: (i, 0)),
        pl.BlockSpec((BN, D), lambda i: (i, 0)),
        _full_spec((8, D)),
        _full_spec((D, D)),
        _full_spec((D, D)),
        _full_spec((D, 8)),
    ],
    out_specs=[
        pl.BlockSpec((BN, D), lambda i: (i, 0)),
        pl.BlockSpec((BN, D), lambda i: (i, 0)),
        pl.BlockSpec((BN, 8), lambda i: (i, 0)),
        pl.BlockSpec((BN, 8), lambda i: (i, 0)),
    ],
    out_shape=[
        jax.ShapeDtypeStruct((NT, D), jnp.float32),
        jax.ShapeDtypeStruct((NT, D), jnp.float32),
        jax.ShapeDtypeStruct((NT, 8), jnp.float32),
        jax.ShapeDtypeStruct((NT, 8), jnp.float32),
    ],
)


def _k3_body(acc_ref, rs_ref, xs_ref, sel_ref, out_ref):
  rs128 = jnp.dot(rs_ref[...], sel_ref[...], preferred_element_type=jnp.float32)
  h = (acc_ref[...] + rs128 * xs_ref[...]) / jnp.maximum(rs128, 1e-12)
  h = jnp.where(h > 0, h, jnp.exp(h) - 1.0)
  out_ref[...] = jnp.where(h > 0, h, jnp.exp(h) - 1.0)


_k3 = pl.pallas_call(
    _k3_body,
    grid=(NT // BN,),
    in_specs=[
        pl.BlockSpec((BN, D), lambda i: (i, 0)),
        pl.BlockSpec((BN, 8), lambda i: (i, 0)),
        pl.BlockSpec((BN, D), lambda i: (i, 0)),
        _full_spec((8, D)),
    ],
    out_specs=pl.BlockSpec((BN, D), lambda i: (i, 0)),
    out_shape=jax.ShapeDtypeStruct((NT, D), jnp.float32),
)


def _to_tiles(table):
  rows = table.shape[0]
  tb = table.astype(jnp.bfloat16).reshape(rows, 2, 16, 2, 2)
  w = lax.bitcast_convert_type(tb, jnp.int16)
  w = lax.bitcast_convert_type(w.reshape(rows, 2, 16, 2, 2), jnp.int32)
  return w.transpose(1, 2, 3, 0).reshape(2, 16, 2 * rows)


def kernel(term_embeddings, relation_embeddings, edge_embed, w1_heads,
           w2_heads, WR, w1_out, w2_out, edge_list, edge_type,
           edge_list_nhop, edge_type_nhop):
  f32 = jnp.float32
  N = term_embeddings.shape[0]
  R = relation_embeddings.shape[0]
  E = edge_list.shape[1]
  EN = edge_list_nhop.shape[1]
  nchn = -(-E // CH)
  nchn += (-nchn) % 3
  nchh = -(-EN // CH)
  nchh += (-nchh) % 3

  xp = jnp.zeros((NT, D), f32).at[:N].set(term_embeddings)
  relp = jnp.zeros((R1, D), f32).at[:R].set(relation_embeddings)

  ws1t = w1_heads[:, :, 0:D].reshape(2 * 64, D).T
  wd1t = w1_heads[:, :, D:2 * D].reshape(2 * 64, D).T
  wr1t = w1_heads[:, :, 2 * D:3 * D].reshape(2 * 64, D).T
  w2blk = jnp.zeros((D, 8), f32)
  w2blk = w2blk.at[0:64, 0].set(w2_heads[0, 0]).at[64:D, 1].set(w2_heads[1, 0])

  wsot = w1_out[:, 0:D].T
  wdot = w1_out[:, D:2 * D].T
  wrot = w1_out[:, 2 * D:3 * D].T
  w2o = jnp.zeros((D, 8), f32).at[:, 0].set(w2_out[0])

  sel2 = jnp.zeros((8, D), f32).at[0, 0:64].set(1.0).at[1, 64:D].set(1.0)
  sel1 = jnp.zeros((8, D), f32).at[0, :].set(1.0)

  i32 = jnp.int32

  def _seg_pack(cols, nreal, nch):
    padn = nch * CH - nreal
    fills = (N, N, R, R)
    rows = [jnp.concatenate([col.astype(i32), jnp.full((padn,), f, i32)])
            for col, f in zip(cols, fills)]
    pack = jnp.stack([r.reshape(nch, CH) for r in rows], axis=1).reshape(-1)
    return jnp.concatenate([pack, jnp.zeros((2 * len(cols) * CH,), i32)])

  packn = _seg_pack([edge_list[0], edge_list[1], edge_type], E, nchn)
  packh = _seg_pack([edge_list_nhop[0], edge_list_nhop[1],
                     edge_type_nhop[:, 0], edge_type_nhop[:, 1]], EN, nchh)

  re1, ar1, outrel, re2, ar2 = _krel(relp, wr1t, w2blk, WR, wrot, w2o)
  xs, xd, as1, ad1 = _k1(xp, ws1t, wd1t, w2blk)

  sc_edge = _build_sc_edge_kernel(nchn, nchh)

  as_t = jnp.stack([as1[:, 0], as1[:, 1]])
  ad_t = jnp.stack([ad1[:, 0], ad1[:, 1]])
  ar_t = jnp.stack([ar1[:, 0], ar1[:, 1]])
  acc1, rs1 = sc_edge(packn, packh, as_t, ad_t, ar_t,
                      _to_tiles(xd), _to_tiles(re1))
  acc1n = acc1.reshape(2, 16, DSUB, NT).transpose(3, 0, 1, 2).reshape(NT, D)
  rs1t = jnp.pad(rs1.T, ((0, 0), (0, 6)))

  xs2, xd2, as2, ad2 = _k2(acc1n, rs1t, xs, sel2, wsot, wdot, w2o)

  as2t = jnp.broadcast_to(as2[:, 0], (2, NT))
  ad2t = jnp.broadcast_to(ad2[:, 0], (2, NT))
  ar2t = jnp.broadcast_to(ar2[:, 0], (2, R1))
  acc2, rs2 = sc_edge(packn, packh, as2t, ad2t, ar2t,
                      _to_tiles(xd2), _to_tiles(re2))
  acc2n = acc2.reshape(2, 16, DSUB, NT).transpose(3, 0, 1, 2).reshape(NT, D)
  rs2t = jnp.pad(rs2[0:1].T, ((0, 0), (0, 7)))

  out_entity = _k3(acc2n, rs2t, xs2, sel1)
  return out_entity[:N], outrel[:R]


# speedup vs baseline: 1.0586x; 1.0586x over previous
import functools

import jax
import jax.numpy as jnp
from jax import lax
from jax.experimental import pallas as pl
from jax.experimental.pallas import tpu as pltpu
from jax.experimental.pallas import tpu_sc as plsc

ALPHA = 0.2
NT = 10240
R1 = 264
CH = 1024
DSUB = 4
BN = 2048
D = 128


@functools.lru_cache(maxsize=None)
def _build_sc_edge_kernel(nchn: int, nchh: int):
  mesh = plsc.VectorSubcoreMesh(core_axis_name="c", subcore_axis_name="s",
                                num_cores=2, num_subcores=16)

  @functools.partial(
      pl.kernel,
      out_type=(jax.ShapeDtypeStruct((2, 16, NT * DSUB), jnp.float32),
                jax.ShapeDtypeStruct((2, NT), jnp.float32)),
      mesh=mesh,
      scratch_types=[
          pltpu.VMEM((NT,), jnp.float32),
          pltpu.VMEM((NT,), jnp.float32),
          pltpu.VMEM((R1,), jnp.float32),
          pltpu.VMEM((NT * 2,), jnp.int32),
          pltpu.VMEM((R1 * 2,), jnp.int32),
          pltpu.VMEM((NT * DSUB,), jnp.float32),
          pltpu.VMEM((NT,), jnp.float32),
          pltpu.VMEM((4 * CH,), jnp.int32),
          pltpu.VMEM((4 * CH,), jnp.int32),
          pltpu.VMEM((4 * CH,), jnp.int32),
          pltpu.SemaphoreType.DMA,
          pltpu.SemaphoreType.DMA,
          pltpu.SemaphoreType.DMA,
      ],
      compiler_params=pltpu.CompilerParams(needs_layout_passes=False))
  def sc_edge(packn_h, packh_h, as_h, ad_h, ar_h, xd_h, re_h,
              acc_o, rs_o,
              as_v, ad_v, ar_v, xd_v, re_v, acc_v, rs_v,
              bufA, bufB, bufC, semA, semB, semC):
    c = lax.axis_index("c")
    s = lax.axis_index("s")

    pltpu.sync_copy(as_h.at[c], as_v)
    pltpu.sync_copy(ad_h.at[c], ad_v)
    pltpu.sync_copy(ar_h.at[c], ar_v)
    pltpu.sync_copy(xd_h.at[c, s], xd_v)
    pltpu.sync_copy(re_h.at[c, s], re_v)

    zero16 = jnp.zeros((16,), jnp.float32)

    def zacc(i, carry):
      for k in range(8):
        acc_v[pl.ds(i * 128 + k * 16, 16)] = zero16
      return carry

    lax.fori_loop(0, NT * DSUB // 128, zacc, 0)

    def zrs(i, carry):
      for k in range(8):
        rs_v[pl.ds(i * 128 + k * 16, 16)] = zero16
      return carry

    lax.fori_loop(0, NT // 128, zrs, 0)

    rs_mask = jnp.broadcast_to(s == 0, (16,))

    def do_group(buf, gb, with_tb):
      e0 = buf[pl.ds(gb, 16)]
      e1 = buf[pl.ds(CH + gb, 16)]
      tA = buf[pl.ds(2 * CH + gb, 16)]
      a = (plsc.load_gather(as_v, [e0]) + plsc.load_gather(ad_v, [e1])
           + plsc.load_gather(ar_v, [tA]))
      if with_tb:
        tB = buf[pl.ds(3 * CH + gb, 16)]
        a = a + plsc.load_gather(ar_v, [tB])
      ee = jnp.exp(jnp.where(a > 0, -a, (-ALPHA) * a))
      plsc.addupdate_scatter(rs_v, [e0], ee, mask=rs_mask)
      himask = jnp.full((16,), -65536, jnp.int32)
      for p in range(2):
        gx = plsc.load_gather(xd_v, [e1 + p * NT])
        gr = plsc.load_gather(re_v, [tA + p * R1])
        v0 = (plsc.bitcast(gx << 16, jnp.float32)
              + plsc.bitcast(gr << 16, jnp.float32))
        v1 = (plsc.bitcast(gx & himask, jnp.float32)
              + plsc.bitcast(gr & himask, jnp.float32))
        if with_tb:
          gb = plsc.load_gather(re_v, [tB + p * R1])
          v0 = v0 + plsc.bitcast(gb << 16, jnp.float32)
          v1 = v1 + plsc.bitcast(gb & himask, jnp.float32)
        plsc.addupdate_scatter(acc_v, [e0 + (2 * p) * NT], ee * v0)
        plsc.addupdate_scatter(acc_v, [e0 + (2 * p + 1) * NT], ee * v1)

    def process_chunk(buf, with_tb):
      @plsc.parallel_loop(0, CH // 16, 1, unroll=4)
      def grp(g):
        do_group(buf, g * 16, with_tb)

    def run_segment(pack_h, nch, wpc, with_tb):
      def issue(buf, sem, ci):
        pltpu.async_copy(pack_h.at[pl.ds(ci * wpc, wpc)],
                         buf.at[pl.ds(0, wpc)], sem)

      def drain(buf, sem):
        pltpu.make_async_copy(pack_h.at[pl.ds(0, wpc)],
                              buf.at[pl.ds(0, wpc)], sem).wait()

      issue(bufA, semA, 0)
      issue(bufB, semB, 1)

      def body(i, carry):
        c = 3 * i
        drain(bufA, semA)
        process_chunk(bufA, with_tb)
        issue(bufC, semC, c + 2)
        drain(bufB, semB)
        process_chunk(bufB, with_tb)
        issue(bufA, semA, c + 3)
        drain(bufC, semC)
        process_chunk(bufC, with_tb)
        issue(bufB, semB, c + 4)
        return carry

      lax.fori_loop(0, nch // 3, body, 0)
      drain(bufA, semA)
      drain(bufB, semB)

    run_segment(packn_h, nchn, 3 * CH, False)
    run_segment(packh_h, nchh, 4 * CH, True)

    pltpu.sync_copy(acc_v, acc_o.at[c, s])

    @pl.when(s == 0)
    def _():
      pltpu.sync_copy(rs_v, rs_o.at[c])

  return sc_edge


def _full_spec(shape):
  return pl.BlockSpec(shape, lambda i: tuple(0 for _ in shape))


def _k1_body(x_ref, wst_ref, wdt_ref, w2_ref,
             rel_ref, wrt1_ref, wr_ref, wrot_ref, w2o_ref,
             xs_ref, xd_ref, as_ref, ad_ref,
             re1_ref, ar1_ref, outrel_ref, re2_ref, ar2_ref):
  x = x_ref[...]
  xs = jnp.dot(x, wst_ref[...], preferred_element_type=jnp.float32)
  xd = jnp.dot(x, wdt_ref[...], preferred_element_type=jnp.float32)
  xs_ref[...] = xs
  xd_ref[...] = xd
  w2 = w2_ref[...]
  as_ref[...] = jnp.dot(xs, w2, preferred_element_type=jnp.float32)
  ad_ref[...] = jnp.dot(xd, w2, preferred_element_type=jnp.float32)

  @pl.when(pl.program_id(0) == 0)
  def _():
    rel = rel_ref[...]
    re1 = jnp.dot(rel, wrt1_ref[...], preferred_element_type=jnp.float32)
    re1_ref[...] = re1
    ar1_ref[...] = jnp.dot(re1, w2, preferred_element_type=jnp.float32)
    outrel = jnp.dot(rel, wr_ref[...], preferred_element_type=jnp.float32)
    outrel_ref[...] = outrel
    re2 = jnp.dot(outrel, wrot_ref[...], preferred_element_type=jnp.float32)
    re2_ref[...] = re2
    ar2_ref[...] = jnp.dot(re2, w2o_ref[...], preferred_element_type=jnp.float32)


_k1 = pl.pallas_call(
    _k1_body,
    grid=(NT // BN,),
    in_specs=[
        pl.BlockSpec((BN, D), lambda i: (i, 0)),
        _full_spec((D, D)),
        _full_spec((D, D)),
        _full_spec((D, 8)),
        _full_spec((R1, D)),
        _full_spec((D, D)),
        _full_spec((D, D)),
        _full_spec((D, D)),
        _full_spec((D, 8)),
    ],
    out_specs=[
        pl.BlockSpec((BN, D), lambda i: (i, 0)),
        pl.BlockSpec((BN, D), lambda i: (i, 0)),
        pl.BlockSpec((BN, 8), lambda i: (i, 0)),
        pl.BlockSpec((BN, 8), lambda i: (i, 0)),
        _full_spec((R1, D)),
        _full_spec((R1, 8)),
        _full_spec((R1, D)),
        _full_spec((R1, D)),
        _full_spec((R1, 8)),
    ],
    out_shape=[
        jax.ShapeDtypeStruct((NT, D), jnp.float32),
        jax.ShapeDtypeStruct((NT, D), jnp.float32),
        jax.ShapeDtypeStruct((NT, 8), jnp.float32),
        jax.ShapeDtypeStruct((NT, 8), jnp.float32),
        jax.ShapeDtypeStruct((R1, D), jnp.float32),
        jax.ShapeDtypeStruct((R1, 8), jnp.float32),
        jax.ShapeDtypeStruct((R1, D), jnp.float32),
        jax.ShapeDtypeStruct((R1, D), jnp.float32),
        jax.ShapeDtypeStruct((R1, 8), jnp.float32),
    ],
)


def _k2_body(acc_ref, rs_ref, xs_ref, sel_ref, wsot_ref, wdot_ref, w2o_ref,
             xs2_ref, xd2_ref, as2_ref, ad2_ref):
  rs128 = jnp.dot(rs_ref[...], sel_ref[...], preferred_element_type=jnp.float32)
  lx = (acc_ref[...] + rs128 * xs_ref[...]) / jnp.maximum(rs128, 1e-12)
  xs2 = jnp.dot(lx, wsot_ref[...], preferred_element_type=jnp.float32)
  xd2 = jnp.dot(lx, wdot_ref[...], preferred_element_type=jnp.float32)
  xs2_ref[...] = xs2
  xd2_ref[...] = xd2
  w2o = w2o_ref[...]
  as2_ref[...] = jnp.dot(xs2, w2o, preferred_element_type=jnp.float32)
  ad2_ref[...] = jnp.dot(xd2, w2o, preferred_element_type=jnp.float32)


_k2 = pl.pallas_call(
    _k2_body,
    grid=(NT // BN,),
    in_specs=[
        pl.BlockSpec((BN, D), lambda i: (i, 0)),
        pl.BlockSpec((BN, 8), lambda i: (i, 0)),
        pl.BlockSpec((BN, D), lambda i: (i, 0)),
        _full_spec((8, D)),
        _full_spec((D, D)),
        _full_spec((D, D)),
        _full_spec((D, 8)),
    ],
    out_specs=[
        pl.BlockSpec((BN, D), lambda i: (i, 0)),
        pl.BlockSpec((BN, D), lambda i: (i, 0)),
        pl.BlockSpec((BN, 8), lambda i: (i, 0)),
        pl.BlockSpec((BN, 8), lambda i: (i, 0)),
    ],
    out_shape=[
        jax.ShapeDtypeStruct((NT, D), jnp.float32),
        jax.ShapeDtypeStruct((NT, D), jnp.float32),
        jax.ShapeDtypeStruct((NT, 8), jnp.float32),
        jax.ShapeDtypeStruct((NT, 8), jnp.float32),
    ],
)


def _k3_body(acc_ref, rs_ref, xs_ref, sel_ref, out_ref):
  rs128 = jnp.dot(rs_ref[...], sel_ref[...], preferred_element_type=jnp.float32)
  h = (acc_ref[...] + rs128 * xs_ref[...]) / jnp.maximum(rs128, 1e-12)
  h = jnp.where(h > 0, h, jnp.exp(h) - 1.0)
  out_ref[...] = jnp.where(h > 0, h, jnp.exp(h) - 1.0)


_k3 = pl.pallas_call(
    _k3_body,
    grid=(NT // BN,),
    in_specs=[
        pl.BlockSpec((BN, D), lambda i: (i, 0)),
        pl.BlockSpec((BN, 8), lambda i: (i, 0)),
        pl.BlockSpec((BN, D), lambda i: (i, 0)),
        _full_spec((8, D)),
    ],
    out_specs=pl.BlockSpec((BN, D), lambda i: (i, 0)),
    out_shape=jax.ShapeDtypeStruct((NT, D), jnp.float32),
)


def _to_tiles(table):
  rows = table.shape[0]
  tb = table.astype(jnp.bfloat16).reshape(rows, 2, 16, 2, 2)
  w = lax.bitcast_convert_type(tb, jnp.int16)
  w = lax.bitcast_convert_type(w.reshape(rows, 2, 16, 2, 2), jnp.int32)
  return w.transpose(1, 2, 3, 0).reshape(2, 16, 2 * rows)


def kernel(term_embeddings, relation_embeddings, edge_embed, w1_heads,
           w2_heads, WR, w1_out, w2_out, edge_list, edge_type,
           edge_list_nhop, edge_type_nhop):
  f32 = jnp.float32
  N = term_embeddings.shape[0]
  R = relation_embeddings.shape[0]
  E = edge_list.shape[1]
  EN = edge_list_nhop.shape[1]
  nchn = -(-E // CH)
  nchn += (-nchn) % 3
  nchh = -(-EN // CH)
  nchh += (-nchh) % 3

  xp = jnp.zeros((NT, D), f32).at[:N].set(term_embeddings)
  relp = jnp.zeros((R1, D), f32).at[:R].set(relation_embeddings)

  ws1t = w1_heads[:, :, 0:D].reshape(2 * 64, D).T
  wd1t = w1_heads[:, :, D:2 * D].reshape(2 * 64, D).T
  wr1t = w1_heads[:, :, 2 * D:3 * D].reshape(2 * 64, D).T
  w2blk = jnp.zeros((D, 8), f32)
  w2blk = w2blk.at[0:64, 0].set(w2_heads[0, 0]).at[64:D, 1].set(w2_heads[1, 0])

  wsot = w1_out[:, 0:D].T
  wdot = w1_out[:, D:2 * D].T
  wrot = w1_out[:, 2 * D:3 * D].T
  w2o = jnp.zeros((D, 8), f32).at[:, 0].set(w2_out[0])

  sel2 = jnp.zeros((8, D), f32).at[0, 0:64].set(1.0).at[1, 64:D].set(1.0)
  sel1 = jnp.zeros((8, D), f32).at[0, :].set(1.0)

  i32 = jnp.int32

  def _seg_pack(cols, nreal, nch):
    padn = nch * CH - nreal
    fills = (N, N, R, R)
    rows = [jnp.concatenate([col.astype(i32), jnp.full((padn,), f, i32)])
            for col, f in zip(cols, fills)]
    pack = jnp.stack([r.reshape(nch, CH) for r in rows], axis=1).reshape(-1)
    return jnp.concatenate([pack, jnp.zeros((2 * len(cols) * CH,), i32)])

  packn = _seg_pack([edge_list[0], edge_list[1], edge_type], E, nchn)
  packh = _seg_pack([edge_list_nhop[0], edge_list_nhop[1],
                     edge_type_nhop[:, 0], edge_type_nhop[:, 1]], EN, nchh)

  (xs, xd, as1, ad1, re1, ar1, outrel, re2, ar2) = _k1(
      xp, ws1t, wd1t, w2blk, relp, wr1t, WR, wrot, w2o)

  sc_edge = _build_sc_edge_kernel(nchn, nchh)

  as_t = jnp.stack([as1[:, 0], as1[:, 1]])
  ad_t = jnp.stack([ad1[:, 0], ad1[:, 1]])
  ar_t = jnp.stack([ar1[:, 0], ar1[:, 1]])
  acc1, rs1 = sc_edge(packn, packh, as_t, ad_t, ar_t,
                      _to_tiles(xd), _to_tiles(re1))
  acc1n = acc1.reshape(2, 16, DSUB, NT).transpose(3, 0, 1, 2).reshape(NT, D)
  rs1t = jnp.pad(rs1.T, ((0, 0), (0, 6)))

  xs2, xd2, as2, ad2 = _k2(acc1n, rs1t, xs, sel2, wsot, wdot, w2o)

  as2t = jnp.broadcast_to(as2[:, 0], (2, NT))
  ad2t = jnp.broadcast_to(ad2[:, 0], (2, NT))
  ar2t = jnp.broadcast_to(ar2[:, 0], (2, R1))
  acc2, rs2 = sc_edge(packn, packh, as2t, ad2t, ar2t,
                      _to_tiles(xd2), _to_tiles(re2))
  acc2n = acc2.reshape(2, 16, DSUB, NT).transpose(3, 0, 1, 2).reshape(NT, D)
  rs2t = jnp.pad(rs2[0:1].T, ((0, 0), (0, 7)))

  out_entity = _k3(acc2n, rs2t, xs2, sel1)
  return out_entity[:N], outrel[:R]
